# Initial kernel scaffold; baseline (speedup 1.0000x reference)
import jax
import jax.numpy as jnp
from jax.experimental import pallas as pl


def kernel(x, edge_index, edge_attr, aminoacids_features, blosum62, idx_batch, cc, monomer_labels, Wnn1, bnn1, root1, b1, Wnn2, bnn2, root2, b2, arma_init_w, arma_w, arma_root_w, arma_bias, W1, bb1, W2, bb2, W3, bb3, W4, bb4):
    raise NotImplementedError("write your pallas kernel here")


# speedup vs baseline: 3.3166x; 3.3166x over previous
import functools

import jax
import jax.numpy as jnp
from jax import lax
from jax.experimental import pallas as pl
from jax.experimental.pallas import tpu as pltpu
from jax.experimental.pallas import tpu_sc as plsc

_N = 10000
_E = 160000
_B = 10
_A = 50
_DIN = 128
_DE = 4
_H = 8
_HG = 64
_K = 3
_T = 10
_NPAD = 10240
_PPAD = 512
_NC, _NS = 2, 16
_NW = _NC * _NS
_FARMA_DIM = _H + 95


def _mm_body(x_ref, w_ref, o_ref):
    o_ref[...] = jnp.dot(x_ref[...], w_ref[...],
                         preferred_element_type=jnp.float32)


def _dense(x, w, block_rows):
    n, k = x.shape
    m = w.shape[1]
    return pl.pallas_call(
        _mm_body,
        grid=(n // block_rows,),
        in_specs=[pl.BlockSpec((block_rows, k), lambda i: (i, 0)),
                  pl.BlockSpec((k, m), lambda i: (0, 0))],
        out_specs=pl.BlockSpec((block_rows, m), lambda i: (i, 0)),
        out_shape=jax.ShapeDtypeStruct((n, m), jnp.float32),
    )(x, w)


def _combine_body(g_ref, ea_ref, o_ref):
    g = g_ref[...]
    ea = ea_ref[...]
    acc = g[:, _DE * _H:(_DE + 1) * _H]
    for dd in range(_DE):
        acc = acc + ea[:, dd:dd + 1] * g[:, dd * _H:(dd + 1) * _H]
    o_ref[...] = acc


def _combine(g, ea, block_rows=2000):
    e = g.shape[0]
    return pl.pallas_call(
        _combine_body,
        grid=(e // block_rows,),
        in_specs=[pl.BlockSpec((block_rows, g.shape[1]), lambda i: (i, 0)),
                  pl.BlockSpec((block_rows, _DE), lambda i: (i, 0))],
        out_specs=pl.BlockSpec((block_rows, _H), lambda i: (i, 0)),
        out_shape=jax.ShapeDtypeStruct((e, _H), jnp.float32),
    )(g, ea)


def _hrelu_mm_body(y_ref, pp_ref, b_ref, w_ref, h_ref, y2_ref):
    h = jnp.maximum(
        y_ref[:, (_DE + 1) * _H:] + pp_ref[0] + pp_ref[1] + b_ref[...], 0.0)
    h_ref[...] = h
    y2_ref[...] = jnp.dot(h, w_ref[...], preferred_element_type=jnp.float32)


def _hrelu_mm(y, pp, b, w, block_rows=1000):
    n, d = y.shape
    m = w.shape[1]
    return pl.pallas_call(
        _hrelu_mm_body,
        grid=(n // block_rows,),
        in_specs=[pl.BlockSpec((block_rows, d), lambda i: (i, 0)),
                  pl.BlockSpec((_NC, block_rows, _H), lambda i: (0, i, 0)),
                  pl.BlockSpec((1, _H), lambda i: (0, 0)),
                  pl.BlockSpec((_H, m), lambda i: (0, 0))],
        out_specs=[pl.BlockSpec((block_rows, _H), lambda i: (i, 0)),
                   pl.BlockSpec((block_rows, m), lambda i: (i, 0))],
        out_shape=[jax.ShapeDtypeStruct((n, _H), jnp.float32),
                   jax.ShapeDtypeStruct((n, m), jnp.float32)],
    )(y, pp, b, w)


def _hrelu_body(y_ref, pp_ref, b_ref, h_ref):
    h_ref[...] = jnp.maximum(
        y_ref[:, (_DE + 1) * _H:] + pp_ref[0] + pp_ref[1] + b_ref[...], 0.0)


def _hrelu(y, pp, b, block_rows=1000):
    n, d = y.shape
    return pl.pallas_call(
        _hrelu_body,
        grid=(n // block_rows,),
        in_specs=[pl.BlockSpec((block_rows, d), lambda i: (i, 0)),
                  pl.BlockSpec((_NC, block_rows, _H), lambda i: (0, i, 0)),
                  pl.BlockSpec((1, _H), lambda i: (0, 0))],
        out_specs=pl.BlockSpec((block_rows, _H), lambda i: (i, 0)),
        out_shape=jax.ShapeDtypeStruct((n, _H), jnp.float32),
    )(y, pp, b)


def _sc_gather(table, idx, chunk=1000):
    e = idx.shape[0]
    d = table.shape[1]
    per_w = e // _NW
    nchunk = per_w // chunk
    mesh = plsc.VectorSubcoreMesh(core_axis_name="c", subcore_axis_name="s")

    @functools.partial(
        pl.kernel,
        out_type=jax.ShapeDtypeStruct((e, d), jnp.float32),
        mesh=mesh,
        scratch_types=[pltpu.VMEM((chunk,), jnp.int32),
                       pltpu.VMEM((chunk, d), jnp.float32),
                       pltpu.SemaphoreType.DMA],
        compiler_params=pltpu.CompilerParams(use_tc_tiling_on_sc=False),
    )
    def k(table_hbm, idx_hbm, out_hbm, idx_v, rows_v, sem):
        wid = lax.axis_index("s") * _NC + lax.axis_index("c")
        base = wid * per_w
        for ci in range(nchunk):
            off = base + ci * chunk
            pltpu.sync_copy(idx_hbm.at[pl.ds(off, chunk)], idx_v)
            pltpu.async_copy(table_hbm.at[idx_v], rows_v, sem).wait()
            pltpu.sync_copy(rows_v, out_hbm.at[pl.ds(off, chunk)])

    return k(table, idx)


def _sc_scatter_add(msg, dst, zeros, npad, chunk):
    e = msg.shape[0]
    d = msg.shape[1]
    per_w = e // _NW
    nchunk = per_w // chunk
    stripe = npad // _NS
    mesh = plsc.VectorSubcoreMesh(core_axis_name="c", subcore_axis_name="s")

    @functools.partial(
        pl.kernel,
        out_type=jax.ShapeDtypeStruct((_NC, npad, d), jnp.float32),
        mesh=mesh,
        scratch_types=[pltpu.VMEM((chunk,), jnp.int32),
                       pltpu.VMEM((chunk, d), jnp.float32),
                       pltpu.VMEM((stripe, d), jnp.float32),
                       pltpu.VMEM_SHARED((npad, d), jnp.float32),
                       pltpu.SemaphoreType.DMA],
        compiler_params=pltpu.CompilerParams(use_tc_tiling_on_sc=False),
    )
    def k(msg_hbm, dst_hbm, zeros_hbm, out_hbm,
          idx_v, msg_v, buf_v, acc_sh, sem):
        cid = lax.axis_index("c")
        sid = lax.axis_index("s")
        wid = sid * _NC + cid
        pltpu.sync_copy(zeros_hbm.at[pl.ds(sid * stripe, stripe)], buf_v)
        pltpu.sync_copy(buf_v, acc_sh.at[pl.ds(sid * stripe, stripe)])
        plsc.subcore_barrier()
        base = wid * per_w
        for ci in range(nchunk):
            off = base + ci * chunk
            pltpu.sync_copy(dst_hbm.at[pl.ds(off, chunk)], idx_v)
            pltpu.sync_copy(msg_hbm.at[pl.ds(off, chunk)], msg_v)
            pltpu.sync_copy(msg_v, acc_sh.at[idx_v], add=True)
        plsc.subcore_barrier()
        pltpu.sync_copy(acc_sh.at[pl.ds(sid * stripe, stripe)], buf_v)
        pltpu.sync_copy(buf_v, out_hbm.at[cid, pl.ds(sid * stripe, stripe)])

    return k(msg, dst, zeros)


def _arma_mlp_body(pp_ref, af_ref, wip_ref, wia_ref, wbd_ref,
                   wrp_ref, wra_ref, bias_ref, sh_ref, sel_ref,
                   w1_ref, b1_ref, w2_ref, b2_ref, w3_ref, b3_ref,
                   w4_ref, b4_ref, o_ref):
    dot = functools.partial(jnp.dot, preferred_element_type=jnp.float32)
    pooled = pp_ref[0] + pp_ref[1]
    af = af_ref[...]
    sh_m = sh_ref[...]
    out = dot(pooled, wip_ref[...]) + dot(af, wia_ref[...])
    for t in range(_T):
        if t > 0:
            out = dot(out, wbd_ref[t - 1])
        root = dot(pooled, wrp_ref[t]) + dot(af, wra_ref[t])
        out = jnp.maximum(dot(sh_m, out) + root + bias_ref[t:t + 1], 0.0)
    m = jnp.maximum(
        (out[:, :_HG] + out[:, _HG:2 * _HG] + out[:, 2 * _HG:]) / 3.0, 0.0)
    p = dot(sel_ref[...], m)
    p = jnp.maximum(dot(p, w1_ref[...]) + b1_ref[...], 0.0)
    p = jnp.maximum(dot(p, w2_ref[...]) + b2_ref[...], 0.0)
    p = jnp.maximum(dot(p, w3_ref[...]) + b3_ref[...], 0.0)
    o_ref[...] = dot(p, w4_ref[...]) + b4_ref[...]


def _arma_mlp(pp, af, wip, wia, wbd, wrp, wra, bias, sh, sel,
              w1, b1, w2, b2, w3, b3, w4, b4):
    return pl.pallas_call(
        _arma_mlp_body,
        out_shape=jax.ShapeDtypeStruct((_B, 1), jnp.float32),
    )(pp, af, wip, wia, wbd, wrp, wra, bias, sh, sel,
      w1, b1, w2, b2, w3, b3, w4, b4)


def _build_wcat(Wnn, bnn, root, din):
    wd = Wnn.reshape(_DE, din, _H).transpose(1, 0, 2).reshape(din, _DE * _H)
    return jnp.concatenate([wd, bnn.reshape(din, _H), root], axis=1)


def kernel(x, edge_index, edge_attr, aminoacids_features, blosum62, idx_batch,
           cc, monomer_labels, Wnn1, bnn1, root1, b1, Wnn2, bnn2, root2, b2,
           arma_init_w, arma_w, arma_root_w, arma_bias,
           W1, bb1, W2, bb2, W3, bb3, W4, bb4):
    src = edge_index[0]
    dst = edge_index[1]
    zeros_n = jnp.zeros((_NPAD, _H), jnp.float32)
    zeros_p = jnp.zeros((_PPAD, _H), jnp.float32)

    wcat1 = _build_wcat(Wnn1, bnn1, root1, _DIN)
    y1 = _dense(x, wcat1, 1000)
    g1 = _sc_gather(y1, src)
    msg1 = _combine(g1, edge_attr)
    agg1 = _sc_scatter_add(msg1, dst, zeros_n, _NPAD, 1000)[:, :_N]

    wcat2 = _build_wcat(Wnn2, bnn2, root2, _H)
    _, y2 = _hrelu_mm(y1, agg1, b1.reshape(1, _H), wcat2)
    g2 = _sc_gather(y2, src)
    msg2 = _combine(g2, edge_attr)
    agg2 = _sc_scatter_add(msg2, dst, zeros_n, _NPAD, 1000)[:, :_N]
    h2 = _hrelu(y2, agg2, b2.reshape(1, _H))

    keys = idx_batch * _A + monomer_labels
    h2p = jnp.concatenate(
        [h2, jnp.zeros((_NPAD - _N, _H), jnp.float32)], axis=0)
    keys_p = jnp.concatenate(
        [keys, jnp.full((_NPAD - _N,), _PPAD - 1, jnp.int32)], axis=0)
    pool = _sc_scatter_add(h2p, keys_p, zeros_p, _PPAD, 320)

    af = aminoacids_features[cc].reshape(_B * _A, 95)
    af = jnp.concatenate(
        [af, jnp.zeros((_PPAD - _B * _A, 95), jnp.float32)], axis=0)
    kh = _K * _HG
    wip = arma_init_w.transpose(1, 0, 2).reshape(_FARMA_DIM, kh)[: _H]
    wia = arma_init_w.transpose(1, 0, 2).reshape(_FARMA_DIM, kh)[_H:]
    wr = arma_root_w.transpose(0, 2, 1, 3).reshape(_T, _FARMA_DIM, kh)
    wrp, wra = wr[:, :_H], wr[:, _H:]
    bias = arma_bias[:, :, 0, :].reshape(_T, kh)
    wbd = jnp.zeros((_T - 1, kh, kh), jnp.float32)
    for k in range(_K):
        wbd = wbd.at[:, k * _HG:(k + 1) * _HG,
                     k * _HG:(k + 1) * _HG].set(arma_w[:, k])
    r = jnp.arange(_PPAD)
    shm = jnp.zeros((_PPAD, _PPAD), jnp.float32).at[
        r[1:], r[1:] - 1].set((r[1:] % _A >= 2).astype(jnp.float32))
    sel = jnp.zeros((_B, _PPAD), jnp.float32).at[
        jnp.arange(_B * _A) // _A, jnp.arange(_B * _A)].set(1.0)

    p = _arma_mlp(pool, af, wip, wia, wbd, wrp, wra, bias, shm, sel,
                  W1, bb1.reshape(1, -1), W2, bb2.reshape(1, -1),
                  W3, bb3.reshape(1, -1), W4, bb4.reshape(1, -1))
    return p.reshape(-1)


# speedup vs baseline: 6.9039x; 2.0817x over previous
import functools

import numpy as np

import jax
import jax.numpy as jnp
from jax import lax
from jax.experimental import pallas as pl
from jax.experimental.pallas import tpu as pltpu
from jax.experimental.pallas import tpu_sc as plsc

_N = 10000
_E = 160000
_B = 10
_A = 50
_DIN = 128
_DE = 4
_H = 8
_HG = 64
_K = 3
_T = 10
_FARMA = _H + 95
_W = (_DE + 2) * _H
_NPAD = 10240
_PPAD = 512
_EP = 163840
_CH = 1024
_NC, _NS = 2, 16
_NW = _NC * _NS


def _mm_body(x_ref, w_ref, o_ref):
    o_ref[...] = jnp.dot(x_ref[...], w_ref[...],
                         preferred_element_type=jnp.float32)


def _dense(x, w, block_rows):
    n, k = x.shape
    m = w.shape[1]
    return pl.pallas_call(
        _mm_body,
        grid=(n // block_rows,),
        in_specs=[pl.BlockSpec((block_rows, k), lambda i: (i, 0)),
                  pl.BlockSpec((k, m), lambda i: (0, 0))],
        out_specs=pl.BlockSpec((block_rows, m), lambda i: (i, 0)),
        out_shape=jax.ShapeDtypeStruct((n, m), jnp.float32),
    )(x, w)


def _hrelu_mm_body(y_ref, pp_ref, b_ref, w_ref, h_ref, y2_ref):
    h = jnp.maximum(
        y_ref[:, (_DE + 1) * _H:] + pp_ref[0] + pp_ref[1] + b_ref[...], 0.0)
    h_ref[...] = h
    y2_ref[...] = jnp.dot(h, w_ref[...], preferred_element_type=jnp.float32)


def _hrelu_mm(y, pp, b, w, block_rows=2000):
    n, d = y.shape
    m = w.shape[1]
    return pl.pallas_call(
        _hrelu_mm_body,
        grid=(n // block_rows,),
        in_specs=[pl.BlockSpec((block_rows, d), lambda i: (i, 0)),
                  pl.BlockSpec((_NC, block_rows, _H), lambda i: (0, i, 0)),
                  pl.BlockSpec((1, _H), lambda i: (0, 0)),
                  pl.BlockSpec((_H, m), lambda i: (0, 0))],
        out_specs=[pl.BlockSpec((block_rows, _H), lambda i: (i, 0)),
                   pl.BlockSpec((block_rows, m), lambda i: (i, 0))],
        out_shape=[jax.ShapeDtypeStruct((n, _H), jnp.float32),
                   jax.ShapeDtypeStruct((n, m), jnp.float32)],
    )(y, pp, b, w)


def _hrelu_body(y_ref, pp_ref, b_ref, h_ref):
    h_ref[...] = jnp.maximum(
        y_ref[:, (_DE + 1) * _H:] + pp_ref[0] + pp_ref[1] + b_ref[...], 0.0)


def _hrelu(y, pp, b, block_rows=2000):
    n, d = y.shape
    return pl.pallas_call(
        _hrelu_body,
        grid=(n // block_rows,),
        in_specs=[pl.BlockSpec((block_rows, d), lambda i: (i, 0)),
                  pl.BlockSpec((_NC, block_rows, _H), lambda i: (0, i, 0)),
                  pl.BlockSpec((1, _H), lambda i: (0, 0))],
        out_specs=pl.BlockSpec((block_rows, _H), lambda i: (i, 0)),
        out_shape=jax.ShapeDtypeStruct((n, _H), jnp.float32),
    )(y, pp, b)


def _sc_edge_layer(table, srcp, dstp, eatp, zeros):
    per_w = _EP // _NW
    nchunk = per_w // _CH
    ngrp = _CH // 16
    stripe = _NPAD // _NS
    mesh = plsc.VectorSubcoreMesh(core_axis_name="c", subcore_axis_name="s")

    @functools.partial(
        pl.kernel,
        out_type=jax.ShapeDtypeStruct((_NC, _NPAD, _H), jnp.float32),
        mesh=mesh,
        scratch_types=[pltpu.VMEM((_CH,), jnp.int32),
                       pltpu.VMEM((_CH,), jnp.int32),
                       pltpu.VMEM((_DE, _CH), jnp.float32),
                       pltpu.VMEM((_CH, _W), jnp.float32),
                       pltpu.VMEM((_CH, _H), jnp.float32),
                       pltpu.VMEM((stripe, _H), jnp.float32),
                       pltpu.VMEM_SHARED((_NPAD, _H), jnp.float32),
                       pltpu.SemaphoreType.DMA],
        compiler_params=pltpu.CompilerParams(use_tc_tiling_on_sc=False,
                                             needs_layout_passes=False),
    )
    def k(table_hbm, src_hbm, dst_hbm, eat_hbm, zeros_hbm, out_hbm,
          srcv, dstv, eav, rows, msg, buf, acc_sh, sem):
        cid = lax.axis_index("c")
        sid = lax.axis_index("s")
        wid = sid * _NC + cid
        pltpu.sync_copy(zeros_hbm.at[pl.ds(sid * stripe, stripe)], buf)
        pltpu.sync_copy(buf, acc_sh.at[pl.ds(sid * stripe, stripe)])
        plsc.subcore_barrier()
        lanes = lax.iota(jnp.int32, 16)
        base_e = wid * per_w
        for ci in range(nchunk):
            off = base_e + ci * _CH
            pltpu.sync_copy(src_hbm.at[pl.ds(off, _CH)], srcv)
            pltpu.sync_copy(dst_hbm.at[pl.ds(off, _CH)], dstv)
            pltpu.sync_copy(eat_hbm.at[:, pl.ds(off, _CH)], eav)
            pltpu.async_copy(table_hbm.at[srcv], rows, sem).wait()

            def grp(g, c):
                b = g * 16
                row16 = lanes + b
                ea = [eav[d, pl.ds(b, 16)] for d in range(_DE)]
                for o in range(_H):
                    acc = plsc.load_gather(
                        rows, [row16, jnp.full((16,), _DE * _H + o, jnp.int32)])
                    for d in range(_DE):
                        acc = acc + ea[d] * plsc.load_gather(
                            rows, [row16, jnp.full((16,), d * _H + o, jnp.int32)])
                    plsc.store_scatter(
                        msg, [row16, jnp.full((16,), o, jnp.int32)], acc)
                return c

            lax.fori_loop(0, ngrp, grp, 0)
            pltpu.sync_copy(msg, acc_sh.at[dstv], add=True)
        plsc.subcore_barrier()
        pltpu.sync_copy(acc_sh.at[pl.ds(sid * stripe, stripe)], buf)
        pltpu.sync_copy(buf, out_hbm.at[cid, pl.ds(sid * stripe, stripe)])

    return k(table, srcp, dstp, eatp, zeros)


def _sc_scatter_add(msg, dst, zeros, npad, chunk):
    e = msg.shape[0]
    d = msg.shape[1]
    per_w = e // _NW
    nchunk = per_w // chunk
    stripe = npad // _NS
    mesh = plsc.VectorSubcoreMesh(core_axis_name="c", subcore_axis_name="s")

    @functools.partial(
        pl.kernel,
        out_type=jax.ShapeDtypeStruct((_NC, npad, d), jnp.float32),
        mesh=mesh,
        scratch_types=[pltpu.VMEM((chunk,), jnp.int32),
                       pltpu.VMEM((chunk, d), jnp.float32),
                       pltpu.VMEM((stripe, d), jnp.float32),
                       pltpu.VMEM_SHARED((npad, d), jnp.float32),
                       pltpu.SemaphoreType.DMA],
        compiler_params=pltpu.CompilerParams(use_tc_tiling_on_sc=False),
    )
    def k(msg_hbm, dst_hbm, zeros_hbm, out_hbm,
          idx_v, msg_v, buf_v, acc_sh, sem):
        cid = lax.axis_index("c")
        sid = lax.axis_index("s")
        wid = sid * _NC + cid
        pltpu.sync_copy(zeros_hbm.at[pl.ds(sid * stripe, stripe)], buf_v)
        pltpu.sync_copy(buf_v, acc_sh.at[pl.ds(sid * stripe, stripe)])
        plsc.subcore_barrier()
        base = wid * per_w
        for ci in range(nchunk):
            off = base + ci * chunk
            pltpu.sync_copy(dst_hbm.at[pl.ds(off, chunk)], idx_v)
            pltpu.sync_copy(msg_hbm.at[pl.ds(off, chunk)], msg_v)
            pltpu.sync_copy(msg_v, acc_sh.at[idx_v], add=True)
        plsc.subcore_barrier()
        pltpu.sync_copy(acc_sh.at[pl.ds(sid * stripe, stripe)], buf_v)
        pltpu.sync_copy(buf_v, out_hbm.at[cid, pl.ds(sid * stripe, stripe)])

    return k(msg, dst, zeros)


def _arma_mlp_body(pp_ref, af_ref, wip_ref, wia_ref, w_ref,
                   wrp_ref, wra_ref, bias_ref, sh_ref, sel_ref,
                   w1_ref, b1_ref, w2_ref, b2_ref, w3_ref, b3_ref,
                   w4_ref, b4_ref, o_ref):
    dot = functools.partial(jnp.dot, preferred_element_type=jnp.float32)
    pooled = pp_ref[0] + pp_ref[1]
    af = af_ref[...]
    sh_m = sh_ref[...]
    out = dot(pooled, wip_ref[...]) + dot(af, wia_ref[...])
    for t in range(_T):
        if t > 0:
            out = jnp.concatenate(
                [dot(out[:, k * _HG:(k + 1) * _HG], w_ref[t - 1, k])
                 for k in range(_K)], axis=1)
        root = dot(pooled, wrp_ref[t]) + dot(af, wra_ref[t])
        out = jnp.maximum(dot(sh_m, out) + root + bias_ref[t:t + 1], 0.0)
    m = jnp.maximum(
        (out[:, :_HG] + out[:, _HG:2 * _HG] + out[:, 2 * _HG:]) / 3.0, 0.0)
    p = dot(sel_ref[...], m)
    p = jnp.maximum(dot(p, w1_ref[...]) + b1_ref[...], 0.0)
    p = jnp.maximum(dot(p, w2_ref[...]) + b2_ref[...], 0.0)
    p = jnp.maximum(dot(p, w3_ref[...]) + b3_ref[...], 0.0)
    o_ref[...] = dot(p, w4_ref[...]) + b4_ref[...]


def _arma_mlp(*args):
    return pl.pallas_call(
        _arma_mlp_body,
        out_shape=jax.ShapeDtypeStruct((_B, 1), jnp.float32),
    )(*args)


_SHM = np.zeros((_PPAD, _PPAD), np.float32)
for _r in range(1, _PPAD):
    if _r % _A >= 2:
        _SHM[_r, _r - 1] = 1.0
_SEL = np.zeros((_B, _PPAD), np.float32)
for _r in range(_B * _A):
    _SEL[_r // _A, _r] = 1.0


def _build_wcat(Wnn, bnn, root, din):
    wd = Wnn.reshape(_DE, din, _H).transpose(1, 0, 2).reshape(din, _DE * _H)
    return jnp.concatenate([wd, bnn.reshape(din, _H), root], axis=1)


def kernel(x, edge_index, edge_attr, aminoacids_features, blosum62, idx_batch,
           cc, monomer_labels, Wnn1, bnn1, root1, b1, Wnn2, bnn2, root2, b2,
           arma_init_w, arma_w, arma_root_w, arma_bias,
           W1, bb1, W2, bb2, W3, bb3, W4, bb4):
    epad = _EP - _E
    srcp = jnp.concatenate([edge_index[0], jnp.zeros((epad,), jnp.int32)])
    dstp = jnp.concatenate(
        [edge_index[1], jnp.full((epad,), _NPAD - 1, jnp.int32)])
    eatp = jnp.concatenate(
        [edge_attr.T, jnp.zeros((_DE, epad), jnp.float32)], axis=1)
    zeros_n = jnp.zeros((_NPAD, _H), jnp.float32)
    zeros_p = jnp.zeros((_PPAD, _H), jnp.float32)

    wcat1 = _build_wcat(Wnn1, bnn1, root1, _DIN)
    y1 = _dense(x, wcat1, 1000)
    agg1 = _sc_edge_layer(y1, srcp, dstp, eatp, zeros_n)[:, :_N]

    wcat2 = _build_wcat(Wnn2, bnn2, root2, _H)
    _, y2 = _hrelu_mm(y1, agg1, b1.reshape(1, _H), wcat2)
    agg2 = _sc_edge_layer(y2, srcp, dstp, eatp, zeros_n)[:, :_N]
    h2 = _hrelu(y2, agg2, b2.reshape(1, _H))

    keys = idx_batch * _A + monomer_labels
    h2p = jnp.concatenate(
        [h2, jnp.zeros((_NPAD - _N, _H), jnp.float32)], axis=0)
    keys_p = jnp.concatenate(
        [keys, jnp.full((_NPAD - _N,), _PPAD - 1, jnp.int32)], axis=0)
    pool = _sc_scatter_add(h2p, keys_p, zeros_p, _PPAD, 320)

    af = aminoacids_features[cc].reshape(_B * _A, 95)
    af = jnp.concatenate(
        [af, jnp.zeros((_PPAD - _B * _A, 95), jnp.float32)], axis=0)
    kh = _K * _HG
    wi = arma_init_w.transpose(1, 0, 2).reshape(_FARMA, kh)
    wr = arma_root_w.transpose(0, 2, 1, 3).reshape(_T, _FARMA, kh)
    bias = arma_bias[:, :, 0, :].reshape(_T, kh)

    p = _arma_mlp(pool, af, wi[:_H], wi[_H:], arma_w, wr[:, :_H], wr[:, _H:],
                  bias, jnp.asarray(_SHM), jnp.asarray(_SEL),
                  W1, bb1.reshape(1, -1), W2, bb2.reshape(1, -1),
                  W3, bb3.reshape(1, -1), W4, bb4.reshape(1, -1))
    return p.reshape(-1)


# speedup vs baseline: 9.2225x; 1.3358x over previous
import functools

import numpy as np

import jax
import jax.numpy as jnp
from jax import lax
from jax.experimental import pallas as pl
from jax.experimental.pallas import tpu as pltpu
from jax.experimental.pallas import tpu_sc as plsc

_N = 10000
_E = 160000
_B = 10
_A = 50
_DIN = 128
_DE = 4
_H = 8
_HG = 64
_K = 3
_T = 10
_FARMA = _H + 95
_W = (_DE + 2) * _H
_NPAD = 10240
_PPAD = 512
_EP = 163840
_CH = 1024
_NC, _NS = 2, 16
_NW = _NC * _NS


def _mm_body(x_ref, w_ref, o_ref):
    o_ref[...] = jnp.dot(x_ref[...], w_ref[...],
                         preferred_element_type=jnp.float32)


def _dense(x, w, block_rows):
    n, k = x.shape
    m = w.shape[1]
    return pl.pallas_call(
        _mm_body,
        grid=(n // block_rows,),
        in_specs=[pl.BlockSpec((block_rows, k), lambda i: (i, 0)),
                  pl.BlockSpec((k, m), lambda i: (0, 0))],
        out_specs=pl.BlockSpec((block_rows, m), lambda i: (i, 0)),
        out_shape=jax.ShapeDtypeStruct((n, m), jnp.float32),
    )(x, w)


def _hrelu_mm_body(y_ref, pp_ref, b_ref, w_ref, h_ref, y2_ref):
    h = jnp.maximum(
        y_ref[:, (_DE + 1) * _H:] + pp_ref[0] + pp_ref[1] + b_ref[...], 0.0)
    h_ref[...] = h
    y2_ref[...] = jnp.dot(h, w_ref[...], preferred_element_type=jnp.float32)


def _hrelu_mm(y, pp, b, w, block_rows=2000):
    n, d = y.shape
    m = w.shape[1]
    return pl.pallas_call(
        _hrelu_mm_body,
        grid=(n // block_rows,),
        in_specs=[pl.BlockSpec((block_rows, d), lambda i: (i, 0)),
                  pl.BlockSpec((_NC, block_rows, _H), lambda i: (0, i, 0)),
                  pl.BlockSpec((1, _H), lambda i: (0, 0)),
                  pl.BlockSpec((_H, m), lambda i: (0, 0))],
        out_specs=[pl.BlockSpec((block_rows, _H), lambda i: (i, 0)),
                   pl.BlockSpec((block_rows, m), lambda i: (i, 0))],
        out_shape=[jax.ShapeDtypeStruct((n, _H), jnp.float32),
                   jax.ShapeDtypeStruct((n, m), jnp.float32)],
    )(y, pp, b, w)


def _hrelu_body(y_ref, pp_ref, b_ref, h_ref):
    h_ref[...] = jnp.maximum(
        y_ref[:, (_DE + 1) * _H:] + pp_ref[0] + pp_ref[1] + b_ref[...], 0.0)


def _hrelu(y, pp, b, block_rows=2000):
    n, d = y.shape
    return pl.pallas_call(
        _hrelu_body,
        grid=(n // block_rows,),
        in_specs=[pl.BlockSpec((block_rows, d), lambda i: (i, 0)),
                  pl.BlockSpec((_NC, block_rows, _H), lambda i: (0, i, 0)),
                  pl.BlockSpec((1, _H), lambda i: (0, 0))],
        out_specs=pl.BlockSpec((block_rows, _H), lambda i: (i, 0)),
        out_shape=jax.ShapeDtypeStruct((n, _H), jnp.float32),
    )(y, pp, b)


def _sc_edge_layer(table, srcp, dstp, eatp, zeros):
    per_w = _EP // _NW
    nchunk = per_w // _CH
    ngrp = _CH // 16
    stripe = _NPAD // _NS
    mesh = plsc.VectorSubcoreMesh(core_axis_name="c", subcore_axis_name="s")

    @functools.partial(
        pl.kernel,
        out_type=jax.ShapeDtypeStruct((_NC, _NPAD, _H), jnp.float32),
        mesh=mesh,
        scratch_types=[pltpu.VMEM((_CH,), jnp.int32),
                       pltpu.VMEM((_CH,), jnp.int32),
                       pltpu.VMEM((_DE, _CH), jnp.float32),
                       pltpu.VMEM((_CH, _W), jnp.float32),
                       pltpu.VMEM((_CH, _H), jnp.float32),
                       pltpu.VMEM((stripe, _H), jnp.float32),
                       pltpu.VMEM_SHARED((_NPAD, _H), jnp.float32),
                       pltpu.SemaphoreType.DMA],
        compiler_params=pltpu.CompilerParams(use_tc_tiling_on_sc=False,
                                             needs_layout_passes=False),
    )
    def k(table_hbm, src_hbm, dst_hbm, eat_hbm, zeros_hbm, out_hbm,
          srcv, dstv, eav, rows, msg, buf, acc_sh, sem):
        cid = lax.axis_index("c")
        sid = lax.axis_index("s")
        wid = sid * _NC + cid
        pltpu.sync_copy(zeros_hbm.at[pl.ds(sid * stripe, stripe)], buf)
        pltpu.sync_copy(buf, acc_sh.at[pl.ds(sid * stripe, stripe)])
        plsc.subcore_barrier()
        lanes = lax.iota(jnp.int32, 16)
        cols = [jnp.full((16,), c, jnp.int32) for c in range((_DE + 1) * _H)]
        base_e = wid * per_w
        for ci in range(nchunk):
            off = base_e + ci * _CH
            pltpu.sync_copy(src_hbm.at[pl.ds(off, _CH)], srcv)
            pltpu.sync_copy(dst_hbm.at[pl.ds(off, _CH)], dstv)
            pltpu.sync_copy(eat_hbm.at[:, pl.ds(off, _CH)], eav)
            pltpu.async_copy(table_hbm.at[srcv], rows, sem).wait()

            @functools.partial(plsc.parallel_loop, 0, ngrp, unroll=2)
            def grp(g):
                b = g * 16
                row16 = lanes + b
                ea = [eav[d, pl.ds(b, 16)] for d in range(_DE)]
                for o in range(_H):
                    ld = [plsc.load_gather(rows, [row16, cols[d * _H + o]])
                          for d in range(_DE + 1)]
                    acc = ((ea[0] * ld[0] + ea[1] * ld[1])
                           + (ea[2] * ld[2] + ea[3] * ld[3]) + ld[4])
                    plsc.store_scatter(msg, [row16, cols[o]], acc)

            pltpu.sync_copy(msg, acc_sh.at[dstv], add=True)
        plsc.subcore_barrier()
        pltpu.sync_copy(acc_sh.at[pl.ds(sid * stripe, stripe)], buf)
        pltpu.sync_copy(buf, out_hbm.at[cid, pl.ds(sid * stripe, stripe)])

    return k(table, srcp, dstp, eatp, zeros)


def _sc_scatter_add(msg, dst, zeros, npad, chunk):
    e = msg.shape[0]
    d = msg.shape[1]
    per_w = e // _NW
    nchunk = per_w // chunk
    stripe = npad // _NS
    mesh = plsc.VectorSubcoreMesh(core_axis_name="c", subcore_axis_name="s")

    @functools.partial(
        pl.kernel,
        out_type=jax.ShapeDtypeStruct((_NC, npad, d), jnp.float32),
        mesh=mesh,
        scratch_types=[pltpu.VMEM((chunk,), jnp.int32),
                       pltpu.VMEM((chunk, d), jnp.float32),
                       pltpu.VMEM((stripe, d), jnp.float32),
                       pltpu.VMEM_SHARED((npad, d), jnp.float32),
                       pltpu.SemaphoreType.DMA],
        compiler_params=pltpu.CompilerParams(use_tc_tiling_on_sc=False),
    )
    def k(msg_hbm, dst_hbm, zeros_hbm, out_hbm,
          idx_v, msg_v, buf_v, acc_sh, sem):
        cid = lax.axis_index("c")
        sid = lax.axis_index("s")
        wid = sid * _NC + cid
        pltpu.sync_copy(zeros_hbm.at[pl.ds(sid * stripe, stripe)], buf_v)
        pltpu.sync_copy(buf_v, acc_sh.at[pl.ds(sid * stripe, stripe)])
        plsc.subcore_barrier()
        base = wid * per_w
        for ci in range(nchunk):
            off = base + ci * chunk
            pltpu.sync_copy(dst_hbm.at[pl.ds(off, chunk)], idx_v)
            pltpu.sync_copy(msg_hbm.at[pl.ds(off, chunk)], msg_v)
            pltpu.sync_copy(msg_v, acc_sh.at[idx_v], add=True)
        plsc.subcore_barrier()
        pltpu.sync_copy(acc_sh.at[pl.ds(sid * stripe, stripe)], buf_v)
        pltpu.sync_copy(buf_v, out_hbm.at[cid, pl.ds(sid * stripe, stripe)])

    return k(msg, dst, zeros)


def _arma_mlp_body(pp_ref, af_ref, wip_ref, wia_ref, w_ref,
                   wrp_ref, wra_ref, bias_ref, sh_ref, sel_ref,
                   w1_ref, b1_ref, w2_ref, b2_ref, w3_ref, b3_ref,
                   w4_ref, b4_ref, o_ref):
    dot = functools.partial(jnp.dot, preferred_element_type=jnp.float32)
    pooled = pp_ref[0] + pp_ref[1]
    af = af_ref[...]
    sh_m = sh_ref[...]
    out = dot(pooled, wip_ref[...]) + dot(af, wia_ref[...])
    for t in range(_T):
        if t > 0:
            out = jnp.concatenate(
                [dot(out[:, k * _HG:(k + 1) * _HG], w_ref[t - 1, k])
                 for k in range(_K)], axis=1)
        root = dot(pooled, wrp_ref[t]) + dot(af, wra_ref[t])
        out = jnp.maximum(dot(sh_m, out) + root + bias_ref[t:t + 1], 0.0)
    m = jnp.maximum(
        (out[:, :_HG] + out[:, _HG:2 * _HG] + out[:, 2 * _HG:]) / 3.0, 0.0)
    p = dot(sel_ref[...], m)
    p = jnp.maximum(dot(p, w1_ref[...]) + b1_ref[...], 0.0)
    p = jnp.maximum(dot(p, w2_ref[...]) + b2_ref[...], 0.0)
    p = jnp.maximum(dot(p, w3_ref[...]) + b3_ref[...], 0.0)
    o_ref[...] = dot(p, w4_ref[...]) + b4_ref[...]


def _arma_mlp(*args):
    return pl.pallas_call(
        _arma_mlp_body,
        out_shape=jax.ShapeDtypeStruct((_B, 1), jnp.float32),
    )(*args)


_SHM = np.zeros((_PPAD, _PPAD), np.float32)
for _r in range(1, _PPAD):
    if _r % _A >= 2:
        _SHM[_r, _r - 1] = 1.0
_SEL = np.zeros((_B, _PPAD), np.float32)
for _r in range(_B * _A):
    _SEL[_r // _A, _r] = 1.0


def _build_wcat(Wnn, bnn, root, din):
    wd = Wnn.reshape(_DE, din, _H).transpose(1, 0, 2).reshape(din, _DE * _H)
    return jnp.concatenate([wd, bnn.reshape(din, _H), root], axis=1)


def kernel(x, edge_index, edge_attr, aminoacids_features, blosum62, idx_batch,
           cc, monomer_labels, Wnn1, bnn1, root1, b1, Wnn2, bnn2, root2, b2,
           arma_init_w, arma_w, arma_root_w, arma_bias,
           W1, bb1, W2, bb2, W3, bb3, W4, bb4):
    epad = _EP - _E
    srcp = jnp.concatenate([edge_index[0], jnp.zeros((epad,), jnp.int32)])
    dstp = jnp.concatenate(
        [edge_index[1], jnp.full((epad,), _NPAD - 1, jnp.int32)])
    eatp = jnp.concatenate(
        [edge_attr.T, jnp.zeros((_DE, epad), jnp.float32)], axis=1)
    zeros_n = jnp.zeros((_NPAD, _H), jnp.float32)
    zeros_p = jnp.zeros((_PPAD, _H), jnp.float32)

    wcat1 = _build_wcat(Wnn1, bnn1, root1, _DIN)
    y1 = _dense(x, wcat1, 1000)
    agg1 = _sc_edge_layer(y1, srcp, dstp, eatp, zeros_n)[:, :_N]

    wcat2 = _build_wcat(Wnn2, bnn2, root2, _H)
    _, y2 = _hrelu_mm(y1, agg1, b1.reshape(1, _H), wcat2)
    agg2 = _sc_edge_layer(y2, srcp, dstp, eatp, zeros_n)[:, :_N]
    h2 = _hrelu(y2, agg2, b2.reshape(1, _H))

    keys = idx_batch * _A + monomer_labels
    h2p = jnp.concatenate(
        [h2, jnp.zeros((_NPAD - _N, _H), jnp.float32)], axis=0)
    keys_p = jnp.concatenate(
        [keys, jnp.full((_NPAD - _N,), _PPAD - 1, jnp.int32)], axis=0)
    pool = _sc_scatter_add(h2p, keys_p, zeros_p, _PPAD, 320)

    af = aminoacids_features[cc].reshape(_B * _A, 95)
    af = jnp.concatenate(
        [af, jnp.zeros((_PPAD - _B * _A, 95), jnp.float32)], axis=0)
    kh = _K * _HG
    wi = arma_init_w.transpose(1, 0, 2).reshape(_FARMA, kh)
    wr = arma_root_w.transpose(0, 2, 1, 3).reshape(_T, _FARMA, kh)
    bias = arma_bias[:, :, 0, :].reshape(_T, kh)

    p = _arma_mlp(pool, af, wi[:_H], wi[_H:], arma_w, wr[:, :_H], wr[:, _H:],
                  bias, jnp.asarray(_SHM), jnp.asarray(_SEL),
                  W1, bb1.reshape(1, -1), W2, bb2.reshape(1, -1),
                  W3, bb3.reshape(1, -1), W4, bb4.reshape(1, -1))
    return p.reshape(-1)


# speedup vs baseline: 9.9918x; 1.0834x over previous
import functools

import numpy as np

import jax
import jax.numpy as jnp
from jax import lax
from jax.experimental import pallas as pl
from jax.experimental.pallas import tpu as pltpu
from jax.experimental.pallas import tpu_sc as plsc

_N = 10000
_E = 160000
_B = 10
_A = 50
_DIN = 128
_DE = 4
_H = 8
_HG = 64
_K = 3
_T = 10
_FARMA = _H + 95
_W = (_DE + 2) * _H
_NPAD = 10240
_PPAD = 512
_EP = 163840
_CH = 1024
_NC, _NS = 2, 16
_NW = _NC * _NS


def _mm_body(x_ref, w_ref, o_ref):
    o_ref[...] = jnp.dot(x_ref[...], w_ref[...],
                         preferred_element_type=jnp.float32)


def _dense(x, w, block_rows):
    n, k = x.shape
    m = w.shape[1]
    return pl.pallas_call(
        _mm_body,
        grid=(n // block_rows,),
        in_specs=[pl.BlockSpec((block_rows, k), lambda i: (i, 0)),
                  pl.BlockSpec((k, m), lambda i: (0, 0))],
        out_specs=pl.BlockSpec((block_rows, m), lambda i: (i, 0)),
        out_shape=jax.ShapeDtypeStruct((n, m), jnp.float32),
    )(x, w)


def _hrelu_mm_body(y_ref, pp_ref, b_ref, w_ref, h_ref, y2_ref):
    h = jnp.maximum(
        y_ref[:, (_DE + 1) * _H:] + pp_ref[0] + pp_ref[1] + b_ref[...], 0.0)
    h_ref[...] = h
    y2_ref[...] = jnp.dot(h, w_ref[...], preferred_element_type=jnp.float32)


def _hrelu_mm(y, pp, b, w, block_rows=2000):
    n, d = y.shape
    m = w.shape[1]
    return pl.pallas_call(
        _hrelu_mm_body,
        grid=(n // block_rows,),
        in_specs=[pl.BlockSpec((block_rows, d), lambda i: (i, 0)),
                  pl.BlockSpec((_NC, block_rows, _H), lambda i: (0, i, 0)),
                  pl.BlockSpec((1, _H), lambda i: (0, 0)),
                  pl.BlockSpec((_H, m), lambda i: (0, 0))],
        out_specs=[pl.BlockSpec((block_rows, _H), lambda i: (i, 0)),
                   pl.BlockSpec((block_rows, m), lambda i: (i, 0))],
        out_shape=[jax.ShapeDtypeStruct((n, _H), jnp.float32),
                   jax.ShapeDtypeStruct((n, m), jnp.float32)],
    )(y, pp, b, w)


def _hrelu_body(y_ref, pp_ref, b_ref, h_ref):
    h_ref[...] = jnp.maximum(
        y_ref[:, (_DE + 1) * _H:] + pp_ref[0] + pp_ref[1] + b_ref[...], 0.0)


def _hrelu(y, pp, b, block_rows=2000):
    n, d = y.shape
    return pl.pallas_call(
        _hrelu_body,
        grid=(n // block_rows,),
        in_specs=[pl.BlockSpec((block_rows, d), lambda i: (i, 0)),
                  pl.BlockSpec((_NC, block_rows, _H), lambda i: (0, i, 0)),
                  pl.BlockSpec((1, _H), lambda i: (0, 0))],
        out_specs=pl.BlockSpec((block_rows, _H), lambda i: (i, 0)),
        out_shape=jax.ShapeDtypeStruct((n, _H), jnp.float32),
    )(y, pp, b)


def _sc_edge_layer(table, srcp, dstp, eatp, zeros):
    per_w = _EP // _NW
    nchunk = per_w // _CH
    ngrp = _CH // 16
    stripe = _NPAD // _NS
    mesh = plsc.VectorSubcoreMesh(core_axis_name="c", subcore_axis_name="s")

    @functools.partial(
        pl.kernel,
        out_type=jax.ShapeDtypeStruct((_NC, _NPAD, _H), jnp.float32),
        mesh=mesh,
        scratch_types=[pltpu.VMEM((_CH,), jnp.int32),
                       pltpu.VMEM((_CH,), jnp.int32),
                       pltpu.VMEM((_DE, _CH), jnp.float32),
                       pltpu.VMEM((_CH, _W), jnp.float32),
                       pltpu.VMEM((_CH, _H), jnp.float32),
                       pltpu.VMEM((stripe, _H), jnp.float32),
                       pltpu.VMEM_SHARED((_NPAD, _H), jnp.float32),
                       pltpu.SemaphoreType.DMA],
        compiler_params=pltpu.CompilerParams(use_tc_tiling_on_sc=False,
                                             needs_layout_passes=False),
    )
    def k(table_hbm, src_hbm, dst_hbm, eat_hbm, zeros_hbm, out_hbm,
          srcv, dstv, eav, rows, msg, buf, acc_sh, sem):
        cid = lax.axis_index("c")
        sid = lax.axis_index("s")
        wid = sid * _NC + cid
        pltpu.sync_copy(zeros_hbm.at[pl.ds(sid * stripe, stripe)], buf)
        pltpu.sync_copy(buf, acc_sh.at[pl.ds(sid * stripe, stripe)])
        plsc.subcore_barrier()
        lanes = lax.iota(jnp.int32, 16)
        cols = [jnp.full((16,), c, jnp.int32) for c in range((_DE + 1) * _H)]
        base_e = wid * per_w
        for ci in range(nchunk):
            off = base_e + ci * _CH
            pltpu.sync_copy(src_hbm.at[pl.ds(off, _CH)], srcv)
            pltpu.sync_copy(dst_hbm.at[pl.ds(off, _CH)], dstv)
            pltpu.sync_copy(eat_hbm.at[:, pl.ds(off, _CH)], eav)
            pltpu.async_copy(table_hbm.at[srcv], rows, sem).wait()

            @functools.partial(plsc.parallel_loop, 0, ngrp, unroll=4)
            def grp(g):
                b = g * 16
                row16 = lanes + b
                ea = [eav[d, pl.ds(b, 16)] for d in range(_DE)]
                for o in range(_H):
                    ld = [plsc.load_gather(rows, [row16, cols[d * _H + o]])
                          for d in range(_DE + 1)]
                    acc = ((ea[0] * ld[0] + ea[1] * ld[1])
                           + (ea[2] * ld[2] + ea[3] * ld[3]) + ld[4])
                    plsc.store_scatter(msg, [row16, cols[o]], acc)

            pltpu.sync_copy(msg, acc_sh.at[dstv], add=True)
        plsc.subcore_barrier()
        pltpu.sync_copy(acc_sh.at[pl.ds(sid * stripe, stripe)], buf)
        pltpu.sync_copy(buf, out_hbm.at[cid, pl.ds(sid * stripe, stripe)])

    return k(table, srcp, dstp, eatp, zeros)


def _sc_scatter_add(msg, dst, zeros, npad, chunk):
    e = msg.shape[0]
    d = msg.shape[1]
    per_w = e // _NW
    nchunk = per_w // chunk
    stripe = npad // _NS
    mesh = plsc.VectorSubcoreMesh(core_axis_name="c", subcore_axis_name="s")

    @functools.partial(
        pl.kernel,
        out_type=jax.ShapeDtypeStruct((_NC, npad, d), jnp.float32),
        mesh=mesh,
        scratch_types=[pltpu.VMEM((chunk,), jnp.int32),
                       pltpu.VMEM((chunk, d), jnp.float32),
                       pltpu.VMEM((stripe, d), jnp.float32),
                       pltpu.VMEM_SHARED((npad, d), jnp.float32),
                       pltpu.SemaphoreType.DMA],
        compiler_params=pltpu.CompilerParams(use_tc_tiling_on_sc=False),
    )
    def k(msg_hbm, dst_hbm, zeros_hbm, out_hbm,
          idx_v, msg_v, buf_v, acc_sh, sem):
        cid = lax.axis_index("c")
        sid = lax.axis_index("s")
        wid = sid * _NC + cid
        pltpu.sync_copy(zeros_hbm.at[pl.ds(sid * stripe, stripe)], buf_v)
        pltpu.sync_copy(buf_v, acc_sh.at[pl.ds(sid * stripe, stripe)])
        plsc.subcore_barrier()
        base = wid * per_w
        for ci in range(nchunk):
            off = base + ci * chunk
            pltpu.sync_copy(dst_hbm.at[pl.ds(off, chunk)], idx_v)
            pltpu.sync_copy(msg_hbm.at[pl.ds(off, chunk)], msg_v)
            pltpu.sync_copy(msg_v, acc_sh.at[idx_v], add=True)
        plsc.subcore_barrier()
        pltpu.sync_copy(acc_sh.at[pl.ds(sid * stripe, stripe)], buf_v)
        pltpu.sync_copy(buf_v, out_hbm.at[cid, pl.ds(sid * stripe, stripe)])

    return k(msg, dst, zeros)


def _arma_mlp_body(pp_ref, af_ref, wip_ref, wia_ref, w_ref,
                   wrp_ref, wra_ref, bias_ref, sh_ref, sel_ref,
                   w1_ref, b1_ref, w2_ref, b2_ref, w3_ref, b3_ref,
                   w4_ref, b4_ref, o_ref):
    dot = functools.partial(jnp.dot, preferred_element_type=jnp.float32)
    pooled = pp_ref[0] + pp_ref[1]
    af = af_ref[...]
    sh_m = sh_ref[...]
    out = dot(pooled, wip_ref[...]) + dot(af, wia_ref[...])
    for t in range(_T):
        if t > 0:
            out = jnp.concatenate(
                [dot(out[:, k * _HG:(k + 1) * _HG], w_ref[t - 1, k])
                 for k in range(_K)], axis=1)
        root = dot(pooled, wrp_ref[t]) + dot(af, wra_ref[t])
        out = jnp.maximum(dot(sh_m, out) + root + bias_ref[t:t + 1], 0.0)
    m = jnp.maximum(
        (out[:, :_HG] + out[:, _HG:2 * _HG] + out[:, 2 * _HG:]) / 3.0, 0.0)
    p = dot(sel_ref[...], m)
    p = jnp.maximum(dot(p, w1_ref[...]) + b1_ref[...], 0.0)
    p = jnp.maximum(dot(p, w2_ref[...]) + b2_ref[...], 0.0)
    p = jnp.maximum(dot(p, w3_ref[...]) + b3_ref[...], 0.0)
    o_ref[...] = dot(p, w4_ref[...]) + b4_ref[...]


def _arma_mlp(*args):
    return pl.pallas_call(
        _arma_mlp_body,
        out_shape=jax.ShapeDtypeStruct((_B, 1), jnp.float32),
    )(*args)


_SHM = np.zeros((_PPAD, _PPAD), np.float32)
for _r in range(1, _PPAD):
    if _r % _A >= 2:
        _SHM[_r, _r - 1] = 1.0
_SEL = np.zeros((_B, _PPAD), np.float32)
for _r in range(_B * _A):
    _SEL[_r // _A, _r] = 1.0


def _build_wcat(Wnn, bnn, root, din):
    wd = Wnn.reshape(_DE, din, _H).transpose(1, 0, 2).reshape(din, _DE * _H)
    return jnp.concatenate([wd, bnn.reshape(din, _H), root], axis=1)


def kernel(x, edge_index, edge_attr, aminoacids_features, blosum62, idx_batch,
           cc, monomer_labels, Wnn1, bnn1, root1, b1, Wnn2, bnn2, root2, b2,
           arma_init_w, arma_w, arma_root_w, arma_bias,
           W1, bb1, W2, bb2, W3, bb3, W4, bb4):
    epad = _EP - _E
    srcp = jnp.concatenate([edge_index[0], jnp.zeros((epad,), jnp.int32)])
    dstp = jnp.concatenate(
        [edge_index[1], jnp.full((epad,), _NPAD - 1, jnp.int32)])
    eatp = jnp.concatenate(
        [edge_attr.T, jnp.zeros((_DE, epad), jnp.float32)], axis=1)
    zeros_n = jnp.zeros((_NPAD, _H), jnp.float32)
    zeros_p = jnp.zeros((_PPAD, _H), jnp.float32)

    wcat1 = _build_wcat(Wnn1, bnn1, root1, _DIN)
    y1 = _dense(x, wcat1, 1000)
    agg1 = _sc_edge_layer(y1, srcp, dstp, eatp, zeros_n)

    wcat2 = _build_wcat(Wnn2, bnn2, root2, _H)
    _, y2 = _hrelu_mm(y1, agg1, b1.reshape(1, _H), wcat2)
    agg2 = _sc_edge_layer(y2, srcp, dstp, eatp, zeros_n)
    h2 = _hrelu(y2, agg2, b2.reshape(1, _H))

    keys = idx_batch * _A + monomer_labels
    h2p = jnp.concatenate(
        [h2, jnp.zeros((_NPAD - _N, _H), jnp.float32)], axis=0)
    keys_p = jnp.concatenate(
        [keys, jnp.full((_NPAD - _N,), _PPAD - 1, jnp.int32)], axis=0)
    pool = _sc_scatter_add(h2p, keys_p, zeros_p, _PPAD, 320)

    af = aminoacids_features[cc].reshape(_B * _A, 95)
    af = jnp.concatenate(
        [af, jnp.zeros((_PPAD - _B * _A, 95), jnp.float32)], axis=0)
    kh = _K * _HG
    wi = arma_init_w.transpose(1, 0, 2).reshape(_FARMA, kh)
    wr = arma_root_w.transpose(0, 2, 1, 3).reshape(_T, _FARMA, kh)
    bias = arma_bias[:, :, 0, :].reshape(_T, kh)

    p = _arma_mlp(pool, af, wi[:_H], wi[_H:], arma_w, wr[:, :_H], wr[:, _H:],
                  bias, jnp.asarray(_SHM), jnp.asarray(_SEL),
                  W1, bb1.reshape(1, -1), W2, bb2.reshape(1, -1),
                  W3, bb3.reshape(1, -1), W4, bb4.reshape(1, -1))
    return p.reshape(-1)


# speedup vs baseline: 10.4309x; 1.0440x over previous
import functools

import numpy as np

import jax
import jax.numpy as jnp
from jax import lax
from jax.experimental import pallas as pl
from jax.experimental.pallas import tpu as pltpu
from jax.experimental.pallas import tpu_sc as plsc

_N = 10000
_E = 160000
_B = 10
_A = 50
_DIN = 128
_DE = 4
_H = 8
_HG = 64
_K = 3
_T = 10
_FARMA = _H + 95
_W = (_DE + 2) * _H
_NPAD = 10240
_PPAD = 512
_EP = 163840
_CH = 512
_NC, _NS = 2, 16
_NW = _NC * _NS


def _mm_body(x_ref, w_ref, o_ref):
    o_ref[...] = jnp.dot(x_ref[...], w_ref[...],
                         preferred_element_type=jnp.float32)


def _dense(x, w, block_rows):
    n, k = x.shape
    m = w.shape[1]
    return pl.pallas_call(
        _mm_body,
        grid=(n // block_rows,),
        in_specs=[pl.BlockSpec((block_rows, k), lambda i: (i, 0)),
                  pl.BlockSpec((k, m), lambda i: (0, 0))],
        out_specs=pl.BlockSpec((block_rows, m), lambda i: (i, 0)),
        out_shape=jax.ShapeDtypeStruct((n, m), jnp.float32),
    )(x, w)


def _hrelu_mm_body(y_ref, pp_ref, b_ref, w_ref, h_ref, y2_ref):
    h = jnp.maximum(
        y_ref[:, (_DE + 1) * _H:] + pp_ref[0] + pp_ref[1] + b_ref[...], 0.0)
    h_ref[...] = h
    y2_ref[...] = jnp.dot(h, w_ref[...], preferred_element_type=jnp.float32)


def _hrelu_mm(y, pp, b, w, block_rows=2000):
    n, d = y.shape
    m = w.shape[1]
    return pl.pallas_call(
        _hrelu_mm_body,
        grid=(n // block_rows,),
        in_specs=[pl.BlockSpec((block_rows, d), lambda i: (i, 0)),
                  pl.BlockSpec((_NC, block_rows, _H), lambda i: (0, i, 0)),
                  pl.BlockSpec((1, _H), lambda i: (0, 0)),
                  pl.BlockSpec((_H, m), lambda i: (0, 0))],
        out_specs=[pl.BlockSpec((block_rows, _H), lambda i: (i, 0)),
                   pl.BlockSpec((block_rows, m), lambda i: (i, 0))],
        out_shape=[jax.ShapeDtypeStruct((n, _H), jnp.float32),
                   jax.ShapeDtypeStruct((n, m), jnp.float32)],
    )(y, pp, b, w)


def _hrelu_body(y_ref, pp_ref, b_ref, h_ref):
    h_ref[...] = jnp.maximum(
        y_ref[:, (_DE + 1) * _H:] + pp_ref[0] + pp_ref[1] + b_ref[...], 0.0)


def _hrelu(y, pp, b, block_rows=2000):
    n, d = y.shape
    return pl.pallas_call(
        _hrelu_body,
        grid=(n // block_rows,),
        in_specs=[pl.BlockSpec((block_rows, d), lambda i: (i, 0)),
                  pl.BlockSpec((_NC, block_rows, _H), lambda i: (0, i, 0)),
                  pl.BlockSpec((1, _H), lambda i: (0, 0))],
        out_specs=pl.BlockSpec((block_rows, _H), lambda i: (i, 0)),
        out_shape=jax.ShapeDtypeStruct((n, _H), jnp.float32),
    )(y, pp, b)


def _sc_edge_layer(table, srcp, dstp, eatp, zeros):
    per_w = _EP // _NW
    nchunk = per_w // _CH
    ngrp = _CH // 16
    stripe = _NPAD // _NS
    mesh = plsc.VectorSubcoreMesh(core_axis_name="c", subcore_axis_name="s")
    vm = pltpu.VMEM

    @functools.partial(
        pl.kernel,
        out_type=jax.ShapeDtypeStruct((_NC, _NPAD, _H), jnp.float32),
        mesh=mesh,
        scratch_types=[[vm((_CH,), jnp.int32)] * 2,
                       [vm((_CH,), jnp.int32)] * 2,
                       [vm((_DE, _CH), jnp.float32)] * 2,
                       [vm((_CH, _W), jnp.float32)] * 2,
                       [vm((_CH, _H), jnp.float32)] * 2,
                       vm((stripe, _H), jnp.float32),
                       pltpu.VMEM_SHARED((_NPAD, _H), jnp.float32),
                       [pltpu.SemaphoreType.DMA] * 2,
                       [pltpu.SemaphoreType.DMA] * 2,
                       [pltpu.SemaphoreType.DMA] * 2],
        compiler_params=pltpu.CompilerParams(use_tc_tiling_on_sc=False,
                                             needs_layout_passes=False),
    )
    def k(table_hbm, src_hbm, dst_hbm, eat_hbm, zeros_hbm, out_hbm,
          srcv, dstv, eav, rows, msg, buf, acc_sh, isem, gsem, ssem):
        cid = lax.axis_index("c")
        sid = lax.axis_index("s")
        wid = sid * _NC + cid
        pltpu.sync_copy(zeros_hbm.at[pl.ds(sid * stripe, stripe)], buf)
        pltpu.sync_copy(buf, acc_sh.at[pl.ds(sid * stripe, stripe)])
        plsc.subcore_barrier()
        lanes = lax.iota(jnp.int32, 16)
        cols = [jnp.full((16,), c, jnp.int32) for c in range((_DE + 1) * _H)]
        base_e = wid * per_w

        def stage(ci, p):
            off = base_e + ci * _CH
            return [
                pltpu.async_copy(src_hbm.at[pl.ds(off, _CH)], srcv[p], isem[p]),
                pltpu.async_copy(dst_hbm.at[pl.ds(off, _CH)], dstv[p], isem[p]),
                pltpu.async_copy(eat_hbm.at[:, pl.ds(off, _CH)], eav[p],
                                 isem[p])]

        st = [None, None]
        st[0] = stage(0, 0)
        for c in st[0]:
            c.wait()
        gcur = pltpu.async_copy(table_hbm.at[srcv[0]], rows[0], gsem[0])
        st[1] = stage(1, 1)
        scat = [None, None]
        for ci in range(nchunk):
            p = ci & 1
            gcur.wait()
            if scat[p] is not None:
                scat[p].wait()
            if ci + 2 < nchunk:
                st[p] = stage(ci + 2, p)
            if ci + 1 < nchunk:
                for c in st[1 - p]:
                    c.wait()
                gnext = pltpu.async_copy(
                    table_hbm.at[srcv[1 - p]], rows[1 - p], gsem[1 - p])

            rows_p, eav_p, msg_p = rows[p], eav[p], msg[p]

            @functools.partial(plsc.parallel_loop, 0, ngrp, unroll=4)
            def grp(g):
                b = g * 16
                row16 = lanes + b
                ea = [eav_p[d, pl.ds(b, 16)] for d in range(_DE)]
                for o in range(_H):
                    ld = [plsc.load_gather(rows_p, [row16, cols[d * _H + o]])
                          for d in range(_DE + 1)]
                    acc = ((ea[0] * ld[0] + ea[1] * ld[1])
                           + (ea[2] * ld[2] + ea[3] * ld[3]) + ld[4])
                    plsc.store_scatter(msg_p, [row16, cols[o]], acc)

            scat[p] = pltpu.async_copy(msg[p], acc_sh.at[dstv[p]], ssem[p],
                                       add=True)
            if ci + 1 < nchunk:
                gcur = gnext
        for p in range(2):
            if scat[p] is not None:
                scat[p].wait()
        plsc.subcore_barrier()
        pltpu.sync_copy(acc_sh.at[pl.ds(sid * stripe, stripe)], buf)
        pltpu.sync_copy(buf, out_hbm.at[cid, pl.ds(sid * stripe, stripe)])

    return k(table, srcp, dstp, eatp, zeros)


def _sc_scatter_add(msg, dst, zeros, npad, chunk):
    e = msg.shape[0]
    d = msg.shape[1]
    per_w = e // _NW
    nchunk = per_w // chunk
    stripe = npad // _NS
    mesh = plsc.VectorSubcoreMesh(core_axis_name="c", subcore_axis_name="s")

    @functools.partial(
        pl.kernel,
        out_type=jax.ShapeDtypeStruct((_NC, npad, d), jnp.float32),
        mesh=mesh,
        scratch_types=[pltpu.VMEM((chunk,), jnp.int32),
                       pltpu.VMEM((chunk, d), jnp.float32),
                       pltpu.VMEM((stripe, d), jnp.float32),
                       pltpu.VMEM_SHARED((npad, d), jnp.float32),
                       pltpu.SemaphoreType.DMA],
        compiler_params=pltpu.CompilerParams(use_tc_tiling_on_sc=False),
    )
    def k(msg_hbm, dst_hbm, zeros_hbm, out_hbm,
          idx_v, msg_v, buf_v, acc_sh, sem):
        cid = lax.axis_index("c")
        sid = lax.axis_index("s")
        wid = sid * _NC + cid
        pltpu.sync_copy(zeros_hbm.at[pl.ds(sid * stripe, stripe)], buf_v)
        pltpu.sync_copy(buf_v, acc_sh.at[pl.ds(sid * stripe, stripe)])
        plsc.subcore_barrier()
        base = wid * per_w
        for ci in range(nchunk):
            off = base + ci * chunk
            pltpu.sync_copy(dst_hbm.at[pl.ds(off, chunk)], idx_v)
            pltpu.sync_copy(msg_hbm.at[pl.ds(off, chunk)], msg_v)
            pltpu.sync_copy(msg_v, acc_sh.at[idx_v], add=True)
        plsc.subcore_barrier()
        pltpu.sync_copy(acc_sh.at[pl.ds(sid * stripe, stripe)], buf_v)
        pltpu.sync_copy(buf_v, out_hbm.at[cid, pl.ds(sid * stripe, stripe)])

    return k(msg, dst, zeros)


def _arma_mlp_body(pp_ref, af_ref, wip_ref, wia_ref, w_ref,
                   wrp_ref, wra_ref, bias_ref, sh_ref, sel_ref,
                   w1_ref, b1_ref, w2_ref, b2_ref, w3_ref, b3_ref,
                   w4_ref, b4_ref, o_ref):
    dot = functools.partial(jnp.dot, preferred_element_type=jnp.float32)
    pooled = pp_ref[0] + pp_ref[1]
    af = af_ref[...]
    sh_m = sh_ref[...]
    out = dot(pooled, wip_ref[...]) + dot(af, wia_ref[...])
    for t in range(_T):
        if t > 0:
            out = jnp.concatenate(
                [dot(out[:, k * _HG:(k + 1) * _HG], w_ref[t - 1, k])
                 for k in range(_K)], axis=1)
        root = dot(pooled, wrp_ref[t]) + dot(af, wra_ref[t])
        out = jnp.maximum(dot(sh_m, out) + root + bias_ref[t:t + 1], 0.0)
    m = jnp.maximum(
        (out[:, :_HG] + out[:, _HG:2 * _HG] + out[:, 2 * _HG:]) / 3.0, 0.0)
    p = dot(sel_ref[...], m)
    p = jnp.maximum(dot(p, w1_ref[...]) + b1_ref[...], 0.0)
    p = jnp.maximum(dot(p, w2_ref[...]) + b2_ref[...], 0.0)
    p = jnp.maximum(dot(p, w3_ref[...]) + b3_ref[...], 0.0)
    o_ref[...] = dot(p, w4_ref[...]) + b4_ref[...]


def _arma_mlp(*args):
    return pl.pallas_call(
        _arma_mlp_body,
        out_shape=jax.ShapeDtypeStruct((_B, 1), jnp.float32),
    )(*args)


_SHM = np.zeros((_PPAD, _PPAD), np.float32)
for _r in range(1, _PPAD):
    if _r % _A >= 2:
        _SHM[_r, _r - 1] = 1.0
_SEL = np.zeros((_B, _PPAD), np.float32)
for _r in range(_B * _A):
    _SEL[_r // _A, _r] = 1.0


def _build_wcat(Wnn, bnn, root, din):
    wd = Wnn.reshape(_DE, din, _H).transpose(1, 0, 2).reshape(din, _DE * _H)
    return jnp.concatenate([wd, bnn.reshape(din, _H), root], axis=1)


def kernel(x, edge_index, edge_attr, aminoacids_features, blosum62, idx_batch,
           cc, monomer_labels, Wnn1, bnn1, root1, b1, Wnn2, bnn2, root2, b2,
           arma_init_w, arma_w, arma_root_w, arma_bias,
           W1, bb1, W2, bb2, W3, bb3, W4, bb4):
    epad = _EP - _E
    srcp = jnp.concatenate([edge_index[0], jnp.zeros((epad,), jnp.int32)])
    dstp = jnp.concatenate(
        [edge_index[1], jnp.full((epad,), _NPAD - 1, jnp.int32)])
    eatp = jnp.concatenate(
        [edge_attr.T, jnp.zeros((_DE, epad), jnp.float32)], axis=1)
    zeros_n = jnp.zeros((_NPAD, _H), jnp.float32)
    zeros_p = jnp.zeros((_PPAD, _H), jnp.float32)

    wcat1 = _build_wcat(Wnn1, bnn1, root1, _DIN)
    y1 = _dense(x, wcat1, 1000)
    agg1 = _sc_edge_layer(y1, srcp, dstp, eatp, zeros_n)

    wcat2 = _build_wcat(Wnn2, bnn2, root2, _H)
    _, y2 = _hrelu_mm(y1, agg1, b1.reshape(1, _H), wcat2)
    agg2 = _sc_edge_layer(y2, srcp, dstp, eatp, zeros_n)
    h2 = _hrelu(y2, agg2, b2.reshape(1, _H))

    keys = idx_batch * _A + monomer_labels
    h2p = jnp.concatenate(
        [h2, jnp.zeros((_NPAD - _N, _H), jnp.float32)], axis=0)
    keys_p = jnp.concatenate(
        [keys, jnp.full((_NPAD - _N,), _PPAD - 1, jnp.int32)], axis=0)
    pool = _sc_scatter_add(h2p, keys_p, zeros_p, _PPAD, 320)

    af = aminoacids_features[cc].reshape(_B * _A, 95)
    af = jnp.concatenate(
        [af, jnp.zeros((_PPAD - _B * _A, 95), jnp.float32)], axis=0)
    kh = _K * _HG
    wi = arma_init_w.transpose(1, 0, 2).reshape(_FARMA, kh)
    wr = arma_root_w.transpose(0, 2, 1, 3).reshape(_T, _FARMA, kh)
    bias = arma_bias[:, :, 0, :].reshape(_T, kh)

    p = _arma_mlp(pool, af, wi[:_H], wi[_H:], arma_w, wr[:, :_H], wr[:, _H:],
                  bias, jnp.asarray(_SHM), jnp.asarray(_SEL),
                  W1, bb1.reshape(1, -1), W2, bb2.reshape(1, -1),
                  W3, bb3.reshape(1, -1), W4, bb4.reshape(1, -1))
    return p.reshape(-1)


# speedup vs baseline: 10.4438x; 1.0012x over previous
import functools

import numpy as np

import jax
import jax.numpy as jnp
from jax import lax
from jax.experimental import pallas as pl
from jax.experimental.pallas import tpu as pltpu
from jax.experimental.pallas import tpu_sc as plsc

_N = 10000
_E = 160000
_B = 10
_A = 50
_DIN = 128
_DE = 4
_H = 8
_HG = 64
_K = 3
_T = 10
_FARMA = _H + 95
_W = (_DE + 2) * _H
_NPAD = 10240
_PPAD = 512
_EP = 163840
_CH = 512
_NC, _NS = 2, 16
_NW = _NC * _NS


def _mm_body(x_ref, w_ref, o_ref):
    o_ref[...] = jnp.dot(x_ref[...], w_ref[...],
                         preferred_element_type=jnp.float32)


def _dense(x, w, block_rows):
    n, k = x.shape
    m = w.shape[1]
    return pl.pallas_call(
        _mm_body,
        grid=(n // block_rows,),
        in_specs=[pl.BlockSpec((block_rows, k), lambda i: (i, 0)),
                  pl.BlockSpec((k, m), lambda i: (0, 0))],
        out_specs=pl.BlockSpec((block_rows, m), lambda i: (i, 0)),
        out_shape=jax.ShapeDtypeStruct((n, m), jnp.float32),
    )(x, w)


def _hrelu_mm_body(y_ref, pp_ref, b_ref, w_ref, h_ref, y2_ref):
    h = jnp.maximum(
        y_ref[:, (_DE + 1) * _H:] + pp_ref[0] + pp_ref[1] + b_ref[...], 0.0)
    h_ref[...] = h
    y2_ref[...] = jnp.dot(h, w_ref[...], preferred_element_type=jnp.float32)


def _hrelu_mm(y, pp, b, w, block_rows=2000):
    n, d = y.shape
    m = w.shape[1]
    return pl.pallas_call(
        _hrelu_mm_body,
        grid=(n // block_rows,),
        in_specs=[pl.BlockSpec((block_rows, d), lambda i: (i, 0)),
                  pl.BlockSpec((_NC, block_rows, _H), lambda i: (0, i, 0)),
                  pl.BlockSpec((1, _H), lambda i: (0, 0)),
                  pl.BlockSpec((_H, m), lambda i: (0, 0))],
        out_specs=[pl.BlockSpec((block_rows, _H), lambda i: (i, 0)),
                   pl.BlockSpec((block_rows, m), lambda i: (i, 0))],
        out_shape=[jax.ShapeDtypeStruct((n, _H), jnp.float32),
                   jax.ShapeDtypeStruct((n, m), jnp.float32)],
    )(y, pp, b, w)


def _hrelu_body(y_ref, pp_ref, b_ref, h_ref):
    h_ref[...] = jnp.maximum(
        y_ref[:, (_DE + 1) * _H:] + pp_ref[0] + pp_ref[1] + b_ref[...], 0.0)


def _hrelu(y, pp, b, block_rows=2000):
    n, d = y.shape
    return pl.pallas_call(
        _hrelu_body,
        grid=(n // block_rows,),
        in_specs=[pl.BlockSpec((block_rows, d), lambda i: (i, 0)),
                  pl.BlockSpec((_NC, block_rows, _H), lambda i: (0, i, 0)),
                  pl.BlockSpec((1, _H), lambda i: (0, 0))],
        out_specs=pl.BlockSpec((block_rows, _H), lambda i: (i, 0)),
        out_shape=jax.ShapeDtypeStruct((n, _H), jnp.float32),
    )(y, pp, b)


def _sc_edge_layer(table, srcp, dstp, eatp, zeros):
    per_w = _EP // _NW
    nchunk = per_w // _CH
    ngrp = _CH // 16
    stripe = _NPAD // _NS
    mesh = plsc.VectorSubcoreMesh(core_axis_name="c", subcore_axis_name="s")
    vm = pltpu.VMEM

    @functools.partial(
        pl.kernel,
        out_type=jax.ShapeDtypeStruct((_NC, _NPAD, _H), jnp.float32),
        mesh=mesh,
        scratch_types=[[vm((_CH,), jnp.int32)] * 4,
                       [vm((_CH,), jnp.int32)] * 4,
                       [vm((_DE, _CH), jnp.float32)] * 4,
                       [vm((_CH, _W), jnp.float32)] * 2,
                       [vm((_CH, _H), jnp.float32)] * 2,
                       vm((stripe, _H), jnp.float32),
                       pltpu.VMEM_SHARED((_NPAD, _H), jnp.float32),
                       [pltpu.SemaphoreType.DMA] * 4,
                       [pltpu.SemaphoreType.DMA] * 2,
                       [pltpu.SemaphoreType.DMA] * 2],
        compiler_params=pltpu.CompilerParams(use_tc_tiling_on_sc=False,
                                             needs_layout_passes=False),
    )
    def k(table_hbm, src_hbm, dst_hbm, eat_hbm, zeros_hbm, out_hbm,
          srcv, dstv, eav, rows, msg, buf, acc_sh, isem, gsem, ssem):
        cid = lax.axis_index("c")
        sid = lax.axis_index("s")
        wid = sid * _NC + cid
        pltpu.sync_copy(zeros_hbm.at[pl.ds(sid * stripe, stripe)], buf)
        pltpu.sync_copy(buf, acc_sh.at[pl.ds(sid * stripe, stripe)])
        plsc.subcore_barrier()
        lanes = lax.iota(jnp.int32, 16)
        cols = [jnp.full((16,), c, jnp.int32) for c in range((_DE + 1) * _H)]
        base_e = wid * per_w

        def stage(ci):
            q = ci % 4
            off = base_e + ci * _CH
            return [
                pltpu.async_copy(src_hbm.at[pl.ds(off, _CH)], srcv[q], isem[q]),
                pltpu.async_copy(dst_hbm.at[pl.ds(off, _CH)], dstv[q], isem[q]),
                pltpu.async_copy(eat_hbm.at[:, pl.ds(off, _CH)], eav[q],
                                 isem[q])]

        st = [None] * 4
        st[0] = stage(0)
        for c in st[0]:
            c.wait()
        gcur = pltpu.async_copy(table_hbm.at[srcv[0]], rows[0], gsem[0])
        st[1] = stage(1)
        scat = [None, None]
        for ci in range(nchunk):
            p = ci & 1
            q = ci % 4
            gcur.wait()
            if scat[p] is not None:
                scat[p].wait()
            if ci + 2 < nchunk:
                st[(ci + 2) % 4] = stage(ci + 2)
            if ci + 1 < nchunk:
                for c in st[(ci + 1) % 4]:
                    c.wait()
                gnext = pltpu.async_copy(
                    table_hbm.at[srcv[(ci + 1) % 4]], rows[1 - p],
                    gsem[1 - p])

            rows_p, eav_p, msg_p = rows[p], eav[q], msg[p]

            @functools.partial(plsc.parallel_loop, 0, ngrp, unroll=4)
            def grp(g):
                b = g * 16
                row16 = lanes + b
                ea = [eav_p[d, pl.ds(b, 16)] for d in range(_DE)]
                for o in range(_H):
                    ld = [plsc.load_gather(rows_p, [row16, cols[d * _H + o]])
                          for d in range(_DE + 1)]
                    acc = ((ea[0] * ld[0] + ea[1] * ld[1])
                           + (ea[2] * ld[2] + ea[3] * ld[3]) + ld[4])
                    plsc.store_scatter(msg_p, [row16, cols[o]], acc)

            scat[p] = pltpu.async_copy(msg[p], acc_sh.at[dstv[q]], ssem[p],
                                       add=True)
            if ci + 1 < nchunk:
                gcur = gnext
        for p in range(2):
            if scat[p] is not None:
                scat[p].wait()
        plsc.subcore_barrier()
        pltpu.sync_copy(acc_sh.at[pl.ds(sid * stripe, stripe)], buf)
        pltpu.sync_copy(buf, out_hbm.at[cid, pl.ds(sid * stripe, stripe)])

    return k(table, srcp, dstp, eatp, zeros)


def _sc_scatter_add(msg, dst, zeros, npad, chunk):
    e = msg.shape[0]
    d = msg.shape[1]
    per_w = e // _NW
    nchunk = per_w // chunk
    stripe = npad // _NS
    mesh = plsc.VectorSubcoreMesh(core_axis_name="c", subcore_axis_name="s")

    @functools.partial(
        pl.kernel,
        out_type=jax.ShapeDtypeStruct((_NC, npad, d), jnp.float32),
        mesh=mesh,
        scratch_types=[pltpu.VMEM((chunk,), jnp.int32),
                       pltpu.VMEM((chunk, d), jnp.float32),
                       pltpu.VMEM((stripe, d), jnp.float32),
                       pltpu.VMEM_SHARED((npad, d), jnp.float32),
                       pltpu.SemaphoreType.DMA],
        compiler_params=pltpu.CompilerParams(use_tc_tiling_on_sc=False),
    )
    def k(msg_hbm, dst_hbm, zeros_hbm, out_hbm,
          idx_v, msg_v, buf_v, acc_sh, sem):
        cid = lax.axis_index("c")
        sid = lax.axis_index("s")
        wid = sid * _NC + cid
        pltpu.sync_copy(zeros_hbm.at[pl.ds(sid * stripe, stripe)], buf_v)
        pltpu.sync_copy(buf_v, acc_sh.at[pl.ds(sid * stripe, stripe)])
        plsc.subcore_barrier()
        base = wid * per_w
        for ci in range(nchunk):
            off = base + ci * chunk
            pltpu.sync_copy(dst_hbm.at[pl.ds(off, chunk)], idx_v)
            pltpu.sync_copy(msg_hbm.at[pl.ds(off, chunk)], msg_v)
            pltpu.sync_copy(msg_v, acc_sh.at[idx_v], add=True)
        plsc.subcore_barrier()
        pltpu.sync_copy(acc_sh.at[pl.ds(sid * stripe, stripe)], buf_v)
        pltpu.sync_copy(buf_v, out_hbm.at[cid, pl.ds(sid * stripe, stripe)])

    return k(msg, dst, zeros)


def _arma_mlp_body(pp_ref, af_ref, wip_ref, wia_ref, w_ref,
                   wrp_ref, wra_ref, bias_ref, sh_ref, sel_ref,
                   w1_ref, b1_ref, w2_ref, b2_ref, w3_ref, b3_ref,
                   w4_ref, b4_ref, o_ref):
    dot = functools.partial(jnp.dot, preferred_element_type=jnp.float32)
    pooled = pp_ref[0] + pp_ref[1]
    af = af_ref[...]
    sh_m = sh_ref[...]
    out = dot(pooled, wip_ref[...]) + dot(af, wia_ref[...])
    for t in range(_T):
        if t > 0:
            out = jnp.concatenate(
                [dot(out[:, k * _HG:(k + 1) * _HG], w_ref[t - 1, k])
                 for k in range(_K)], axis=1)
        root = dot(pooled, wrp_ref[t]) + dot(af, wra_ref[t])
        out = jnp.maximum(dot(sh_m, out) + root + bias_ref[t:t + 1], 0.0)
    m = jnp.maximum(
        (out[:, :_HG] + out[:, _HG:2 * _HG] + out[:, 2 * _HG:]) / 3.0, 0.0)
    p = dot(sel_ref[...], m)
    p = jnp.maximum(dot(p, w1_ref[...]) + b1_ref[...], 0.0)
    p = jnp.maximum(dot(p, w2_ref[...]) + b2_ref[...], 0.0)
    p = jnp.maximum(dot(p, w3_ref[...]) + b3_ref[...], 0.0)
    o_ref[...] = dot(p, w4_ref[...]) + b4_ref[...]


def _arma_mlp(*args):
    return pl.pallas_call(
        _arma_mlp_body,
        out_shape=jax.ShapeDtypeStruct((_B, 1), jnp.float32),
    )(*args)


_SHM = np.zeros((_PPAD, _PPAD), np.float32)
for _r in range(1, _PPAD):
    if _r % _A >= 2:
        _SHM[_r, _r - 1] = 1.0
_SEL = np.zeros((_B, _PPAD), np.float32)
for _r in range(_B * _A):
    _SEL[_r // _A, _r] = 1.0


def _build_wcat(Wnn, bnn, root, din):
    wd = Wnn.reshape(_DE, din, _H).transpose(1, 0, 2).reshape(din, _DE * _H)
    return jnp.concatenate([wd, bnn.reshape(din, _H), root], axis=1)


def kernel(x, edge_index, edge_attr, aminoacids_features, blosum62, idx_batch,
           cc, monomer_labels, Wnn1, bnn1, root1, b1, Wnn2, bnn2, root2, b2,
           arma_init_w, arma_w, arma_root_w, arma_bias,
           W1, bb1, W2, bb2, W3, bb3, W4, bb4):
    epad = _EP - _E
    srcp = jnp.concatenate([edge_index[0], jnp.zeros((epad,), jnp.int32)])
    dstp = jnp.concatenate(
        [edge_index[1], jnp.full((epad,), _NPAD - 1, jnp.int32)])
    eatp = jnp.concatenate(
        [edge_attr.T, jnp.zeros((_DE, epad), jnp.float32)], axis=1)
    zeros_n = jnp.zeros((_NPAD, _H), jnp.float32)
    zeros_p = jnp.zeros((_PPAD, _H), jnp.float32)

    wcat1 = _build_wcat(Wnn1, bnn1, root1, _DIN)
    y1 = _dense(x, wcat1, 1000)
    agg1 = _sc_edge_layer(y1, srcp, dstp, eatp, zeros_n)

    wcat2 = _build_wcat(Wnn2, bnn2, root2, _H)
    _, y2 = _hrelu_mm(y1, agg1, b1.reshape(1, _H), wcat2)
    agg2 = _sc_edge_layer(y2, srcp, dstp, eatp, zeros_n)
    h2 = _hrelu(y2, agg2, b2.reshape(1, _H))

    keys = idx_batch * _A + monomer_labels
    h2p = jnp.concatenate(
        [h2, jnp.zeros((_NPAD - _N, _H), jnp.float32)], axis=0)
    keys_p = jnp.concatenate(
        [keys, jnp.full((_NPAD - _N,), _PPAD - 1, jnp.int32)], axis=0)
    pool = _sc_scatter_add(h2p, keys_p, zeros_p, _PPAD, 320)

    af = aminoacids_features[cc].reshape(_B * _A, 95)
    af = jnp.concatenate(
        [af, jnp.zeros((_PPAD - _B * _A, 95), jnp.float32)], axis=0)
    kh = _K * _HG
    wi = arma_init_w.transpose(1, 0, 2).reshape(_FARMA, kh)
    wr = arma_root_w.transpose(0, 2, 1, 3).reshape(_T, _FARMA, kh)
    bias = arma_bias[:, :, 0, :].reshape(_T, kh)

    p = _arma_mlp(pool, af, wi[:_H], wi[_H:], arma_w, wr[:, :_H], wr[:, _H:],
                  bias, jnp.asarray(_SHM), jnp.asarray(_SEL),
                  W1, bb1.reshape(1, -1), W2, bb2.reshape(1, -1),
                  W3, bb3.reshape(1, -1), W4, bb4.reshape(1, -1))
    return p.reshape(-1)
